# baseline (speedup 1.0000x reference)
import jax
import jax.numpy as jnp
from jax import lax
from jax.experimental import pallas as pl
from jax.experimental.pallas import tpu as pltpu
from jax.experimental.pallas import tpu_sc as plsc

VOCAB = 1000000
MAXLEN = 200
EMBED = 32
BATCH = 4096

NC, NS, L = 2, 16, 16
NW = NC * NS
BPW = BATCH // NW
T_CH = 8
NCHUNK = MAXLEN // T_CH
EB = EMBED // 8
TILE_F = 8 * 128


def _body(xT_hbm, tab_hbm, pos_hbm, out_hbm,
          idx0_v, idx1_v, rows0_v, rows1_v, outv_v, pos_v,
          sem_g0, sem_g1, sem_w):
    wid = lax.axis_index("s") * NC + lax.axis_index("c")
    col0 = pl.multiple_of(wid * BPW, BPW)

    idxb = (idx0_v, idx1_v)
    rows = (rows0_v, rows1_v)
    sem_g = (sem_g0, sem_g1)

    pltpu.sync_copy(pos_hbm, pos_v)

    e16 = lax.iota(jnp.int32, 16)
    ip0 = ((e16 >> 3) << 10) + ((e16 & 7) << 7)
    ip1 = ip0 + 2048

    def issue_chunk(b, c):
        pltpu.sync_copy(
            xT_hbm.at[pl.ds(pl.multiple_of(c * T_CH, T_CH), T_CH),
                      pl.ds(col0, BPW)],
            idxb[b],
        )
        for k in range(T_CH):
            pltpu.async_copy(
                tab_hbm.at[idxb[b].at[k]],
                rows[b].at[pl.ds(k * BPW, BPW)],
                sem_g[b],
            )

    def drain_gathers(b):
        for _ in range(T_CH):
            pltpu.make_async_copy(
                tab_hbm.at[pl.ds(0, BPW)], rows[b].at[pl.ds(0, BPW)], sem_g[b]
            ).wait()

    def drain_out(n):
        for _ in range(n):
            pltpu.make_async_copy(
                outv_v.at[pl.ds(0, TILE_F)], out_hbm.at[0, 0, 0], sem_w
            ).wait()

    def transpose_chunk(b, c):
        for k in range(T_CH):
            t = c * T_CH + k
            pv0 = pos_v[t, pl.ds(0, L)]
            pv1 = pos_v[t, pl.ds(L, L)]
            ipk0 = ip0 + k * 4096
            ipk1 = ip1 + k * 4096

            @plsc.parallel_loop(0, BPW, unroll=8)
            def row_body(bl):
                r = k * BPW + bl
                v0 = rows[b][r, pl.ds(0, L)] + pv0
                v1 = rows[b][r, pl.ds(L, L)] + pv1
                plsc.store_scatter(outv_v, [ipk0 + bl], v0)
                plsc.store_scatter(outv_v, [ipk1 + bl], v1)

    def issue_out(c):
        for k in range(T_CH):
            t = c * T_CH + k
            for i in range(EB):
                pltpu.async_copy(
                    outv_v.at[pl.ds((k * EB + i) * 1024, 1024)],
                    out_hbm.at[t, i, wid],
                    sem_w,
                )

    issue_chunk(0, 0)
    issue_chunk(1, 1)

    def pair_body(p, _):
        c0 = 2 * p
        c1 = c0 + 1
        drain_gathers(0)

        @pl.when(p > 0)
        def _():
            drain_out(T_CH * EB)

        transpose_chunk(0, c0)
        issue_out(c0)
        issue_chunk(0, c0 + 2)
        drain_gathers(1)
        drain_out(T_CH * EB)
        transpose_chunk(1, c1)
        issue_out(c1)

        @pl.when(p < 11)
        def _():
            issue_chunk(1, c1 + 2)

        return 0

    lax.fori_loop(0, 12, pair_body, 0)
    drain_gathers(0)
    drain_out(T_CH * EB)
    transpose_chunk(0, 24)
    issue_out(24)
    drain_out(T_CH * EB)


NTCOL = VOCAB // 128
TAIL0 = NTCOL * 128
GCOL = 6
NGRP = NTCOL // GCOL


def _conv_body(tabT_hbm, tail_hbm, out_hbm,
               in0_v, in1_v, o0_v, o1_v, tail_v,
               sem_i0, sem_i1, sem_o0, sem_o1):
    wid = lax.axis_index("s") * NC + lax.axis_index("c")
    ng = jnp.where(wid < NGRP - (NGRP // NW) * NW, NGRP // NW + 1, NGRP // NW)

    in_v = (in0_v, in1_v)
    out_v = (o0_v, o1_v)
    sem_i = (sem_i0, sem_i1)
    sem_o = (sem_o0, sem_o1)

    lane32 = lax.iota(jnp.int32, 16) * 32

    def grp(j):
        return wid + NW * j

    def issue_in(b, j):
        c0 = grp(j) * GCOL
        for i in range(4):
            pltpu.async_copy(
                tabT_hbm.at[pl.ds(8 * i, 8),
                            pl.ds(pl.multiple_of(c0 * 128, 128), GCOL * 128)],
                in_v[b].at[pl.ds(8 * i, 8)],
                sem_i[b],
            )

    def drain_in(b):
        for i in range(4):
            pltpu.make_async_copy(
                tabT_hbm.at[pl.ds(0, 8), pl.ds(0, GCOL * 128)],
                in_v[b].at[pl.ds(0, 8)],
                sem_i[b],
            ).wait()

    def issue_out(b, j):
        c0 = grp(j) * GCOL
        pltpu.async_copy(
            out_v[b],
            out_hbm.at[pl.ds(pl.multiple_of(c0 * 4096, 8), GCOL * 4096)],
            sem_o[b],
        )

    def drain_out(b):
        pltpu.make_async_copy(out_v[b], out_hbm.at[pl.ds(0, GCOL * 4096)],
                              sem_o[b]).wait()

    def transpose(b):
        @plsc.parallel_loop(0, 32, unroll=2)
        def _(e):
            for tg in range(8 * GCOL):
                v = in_v[b][e, pl.ds(tg * L, L)]
                plsc.store_scatter(out_v[b], [lane32 + (tg * 512 + e)], v)

    issue_in(0, 0)
    issue_in(1, 1)

    def pair_body(p, _):
        for b in range(2):
            j = 2 * p + b
            drain_in(b)

            @pl.when(p > 0)
            def _():
                drain_out(b)

            transpose(b)
            issue_out(b, j)

            @pl.when(j + 2 < ng)
            def _():
                issue_in(b, j + 2)

        return 0

    lax.fori_loop(0, ng // 2, pair_body, 0)

    @pl.when(ng % 2 == 1)
    def _():
        drain_in(0)
        drain_out(0)
        transpose(0)
        issue_out(0, ng - 1)

    @pl.when(wid == 0)
    def _():
        pltpu.sync_copy(tail_hbm, tail_v)
        pltpu.sync_copy(
            tail_v, out_hbm.at[pl.ds((VOCAB - 64) * EMBED, 64 * EMBED)]
        )

    drain_out(0)
    drain_out(1)


def kernel(x, token_table, pos_table):
    xT = x.astype(jnp.int32).T
    mesh = plsc.VectorSubcoreMesh(core_axis_name="c", subcore_axis_name="s")
    conv = pl.kernel(
        _conv_body,
        mesh=mesh,
        out_type=jax.ShapeDtypeStruct((VOCAB * EMBED,), jnp.float32),
        scratch_types=(
            [pltpu.VMEM((32, GCOL * 128), jnp.float32)] * 2
            + [pltpu.VMEM((GCOL * 4096,), jnp.float32)] * 2
            + [pltpu.VMEM((64 * EMBED,), jnp.float32)]
            + [pltpu.SemaphoreType.DMA] * 4
        ),
        compiler_params=pltpu.CompilerParams(
            use_tc_tiling_on_sc=True, needs_layout_passes=False
        ),
    )
    tail = token_table[TAIL0:].reshape(-1)
    tabL = conv(token_table.T, tail).reshape(VOCAB, EMBED)
    run = pl.kernel(
        _body,
        mesh=mesh,
        out_type=jax.ShapeDtypeStruct((MAXLEN, EB, NW, 1024), jnp.float32),
        scratch_types=[
            pltpu.VMEM((T_CH, BPW), jnp.int32),
            pltpu.VMEM((T_CH, BPW), jnp.int32),
            pltpu.VMEM((T_CH * BPW, EMBED), jnp.float32),
            pltpu.VMEM((T_CH * BPW, EMBED), jnp.float32),
            pltpu.VMEM((T_CH * EB * 1024,), jnp.float32),
            pltpu.VMEM((MAXLEN, EMBED), jnp.float32),
            pltpu.SemaphoreType.DMA,
            pltpu.SemaphoreType.DMA,
            pltpu.SemaphoreType.DMA,
        ],
        compiler_params=pltpu.CompilerParams(
            use_tc_tiling_on_sc=False, needs_layout_passes=False
        ),
    )
    out4 = run(xT, tabL, pos_table)
    out5 = out4.reshape(MAXLEN, EB, NW, 8, 128)
    return out5.transpose(2, 4, 0, 1, 3).reshape(BATCH, MAXLEN, EMBED)


# speedup vs baseline: 1.0085x; 1.0085x over previous
import jax
import jax.numpy as jnp
from jax import lax
from jax.experimental import pallas as pl
from jax.experimental.pallas import tpu as pltpu
from jax.experimental.pallas import tpu_sc as plsc

VOCAB = 1000000
MAXLEN = 200
EMBED = 32
BATCH = 4096

NC, NS, L = 2, 16, 16
NW = NC * NS
BPW = BATCH // NW
T_CH = 8
NCHUNK = MAXLEN // T_CH
EB = EMBED // 8
TILE_F = 8 * 128


def _body(xT_hbm, tab_hbm, pos_hbm, out_hbm,
          idx0_v, idx1_v, rows0_v, rows1_v, outv_v, pos_v,
          sem_g0, sem_g1, sem_w):
    wid = lax.axis_index("s") * NC + lax.axis_index("c")
    col0 = pl.multiple_of(wid * BPW, BPW)

    idxb = (idx0_v, idx1_v)
    rows = (rows0_v, rows1_v)
    sem_g = (sem_g0, sem_g1)

    pltpu.sync_copy(pos_hbm, pos_v)

    e16 = lax.iota(jnp.int32, 16)
    ip0 = ((e16 >> 3) << 10) + ((e16 & 7) << 7)
    ip1 = ip0 + 2048

    def issue_chunk(b, c):
        pltpu.sync_copy(
            xT_hbm.at[pl.ds(pl.multiple_of(c * T_CH, T_CH), T_CH),
                      pl.ds(col0, BPW)],
            idxb[b],
        )
        for k in range(T_CH):
            pltpu.async_copy(
                tab_hbm.at[idxb[b].at[k]],
                rows[b].at[pl.ds(k * BPW, BPW)],
                sem_g[b],
            )

    def drain_gathers(b):
        for _ in range(T_CH):
            pltpu.make_async_copy(
                tab_hbm.at[pl.ds(0, BPW)], rows[b].at[pl.ds(0, BPW)], sem_g[b]
            ).wait()

    def drain_out(n):
        for _ in range(n):
            pltpu.make_async_copy(
                outv_v.at[pl.ds(0, TILE_F)], out_hbm.at[0, 0, 0], sem_w
            ).wait()

    def transpose_chunk(b, c):
        for k in range(T_CH):
            t = c * T_CH + k
            pv0 = pos_v[t, pl.ds(0, L)]
            pv1 = pos_v[t, pl.ds(L, L)]
            ipk0 = ip0 + k * 4096
            ipk1 = ip1 + k * 4096

            @plsc.parallel_loop(0, BPW, unroll=8)
            def row_body(bl):
                r = k * BPW + bl
                v0 = rows[b][r, pl.ds(0, L)] + pv0
                v1 = rows[b][r, pl.ds(L, L)] + pv1
                plsc.store_scatter(outv_v, [ipk0 + bl], v0)
                plsc.store_scatter(outv_v, [ipk1 + bl], v1)

    def issue_out(c):
        for k in range(T_CH):
            t = c * T_CH + k
            for i in range(EB):
                pltpu.async_copy(
                    outv_v.at[pl.ds((k * EB + i) * 1024, 1024)],
                    out_hbm.at[t, i, wid],
                    sem_w,
                )

    issue_chunk(0, 0)
    issue_chunk(1, 1)

    def pair_body(p, _):
        c0 = 2 * p
        c1 = c0 + 1
        drain_gathers(0)

        @pl.when(p > 0)
        def _():
            drain_out(T_CH * EB)

        transpose_chunk(0, c0)
        issue_out(c0)
        issue_chunk(0, c0 + 2)
        drain_gathers(1)
        drain_out(T_CH * EB)
        transpose_chunk(1, c1)
        issue_out(c1)

        @pl.when(p < 11)
        def _():
            issue_chunk(1, c1 + 2)

        return 0

    lax.fori_loop(0, 12, pair_body, 0)
    drain_gathers(0)
    drain_out(T_CH * EB)
    transpose_chunk(0, 24)
    issue_out(24)
    drain_out(T_CH * EB)


NTCOL = VOCAB // 128
TAIL0 = NTCOL * 128


def _conv_body(tabT_hbm, tail_hbm, out_hbm,
               in0_v, in1_v, in2_v, in3_v, o0_v, o1_v, o2_v, o3_v, tail_v,
               sem_i0, sem_i1, sem_i2, sem_i3,
               sem_o0, sem_o1, sem_o2, sem_o3):
    wid = lax.axis_index("s") * NC + lax.axis_index("c")
    nc = jnp.where(wid < NTCOL - (NTCOL // NW) * NW, NTCOL // NW + 1, NTCOL // NW)

    in_v = (in0_v, in1_v, in2_v, in3_v)
    out_v = (o0_v, o1_v, o2_v, o3_v)
    sem_i = (sem_i0, sem_i1, sem_i2, sem_i3)
    sem_o = (sem_o0, sem_o1, sem_o2, sem_o3)

    lane32 = lax.iota(jnp.int32, 16) * 32

    def col(j):
        return wid + NW * j

    def issue_in(b, j):
        c = col(j)
        pltpu.async_copy(
            tabT_hbm.at[pl.ds(0, 32), pl.ds(pl.multiple_of(c * 128, 128), 128)],
            in_v[b],
            sem_i[b],
        )

    def drain_in(b):
        pltpu.make_async_copy(
            tabT_hbm.at[pl.ds(0, 32), pl.ds(0, 128)], in_v[b], sem_i[b]
        ).wait()

    def issue_out(b, j):
        c = col(j)
        pltpu.async_copy(
            out_v[b], out_hbm.at[pl.ds(pl.multiple_of(c * 4096, 8), 4096)], sem_o[b]
        )

    def drain_out(b):
        pltpu.make_async_copy(
            out_v[b], out_hbm.at[pl.ds(0, 4096)], sem_o[b]
        ).wait()

    def transpose(b):
        @plsc.parallel_loop(0, 32, unroll=4)
        def _(e):
            for t0 in range(0, 128, 16):
                v = in_v[b][e, pl.ds(t0, L)]
                plsc.store_scatter(out_v[b], [lane32 + (t0 * 32 + e)], v)

    for b in range(4):
        issue_in(b, b)

    def quad_body(p, _):
        for b in range(4):
            j = 4 * p + b
            drain_in(b)

            @pl.when(p > 0)
            def _():
                drain_out(b)

            transpose(b)
            issue_out(b, j)

            @pl.when(j + 4 < nc)
            def _():
                issue_in(b, j + 4)

        return 0

    lax.fori_loop(0, nc // 4, quad_body, 0)
    for r in range(3):
        @pl.when(r < nc % 4)
        def _():
            drain_in(r)
            drain_out(r)
            transpose(r)
            issue_out(r, (nc // 4) * 4 + r)

    @pl.when(wid == 0)
    def _():
        pltpu.sync_copy(tail_hbm, tail_v)
        pltpu.sync_copy(
            tail_v, out_hbm.at[pl.ds((VOCAB - 64) * EMBED, 64 * EMBED)]
        )

    for b in range(4):
        drain_out(b)


def kernel(x, token_table, pos_table):
    xT = x.astype(jnp.int32).T
    mesh = plsc.VectorSubcoreMesh(core_axis_name="c", subcore_axis_name="s")
    conv = pl.kernel(
        _conv_body,
        mesh=mesh,
        out_type=jax.ShapeDtypeStruct((VOCAB * EMBED,), jnp.float32),
        scratch_types=(
            [pltpu.VMEM((32, 128), jnp.float32)] * 4
            + [pltpu.VMEM((4096,), jnp.float32)] * 4
            + [pltpu.VMEM((64 * EMBED,), jnp.float32)]
            + [pltpu.SemaphoreType.DMA] * 8
        ),
        compiler_params=pltpu.CompilerParams(
            use_tc_tiling_on_sc=True, needs_layout_passes=False
        ),
    )
    tail = token_table[TAIL0:].reshape(-1)
    tabL = conv(token_table.T, tail).reshape(VOCAB, EMBED)
    run = pl.kernel(
        _body,
        mesh=mesh,
        out_type=jax.ShapeDtypeStruct((MAXLEN, EB, NW, 1024), jnp.float32),
        scratch_types=[
            pltpu.VMEM((T_CH, BPW), jnp.int32),
            pltpu.VMEM((T_CH, BPW), jnp.int32),
            pltpu.VMEM((T_CH * BPW, EMBED), jnp.float32),
            pltpu.VMEM((T_CH * BPW, EMBED), jnp.float32),
            pltpu.VMEM((T_CH * EB * 1024,), jnp.float32),
            pltpu.VMEM((MAXLEN, EMBED), jnp.float32),
            pltpu.SemaphoreType.DMA,
            pltpu.SemaphoreType.DMA,
            pltpu.SemaphoreType.DMA,
        ],
        compiler_params=pltpu.CompilerParams(
            use_tc_tiling_on_sc=False, needs_layout_passes=False
        ),
    )
    out4 = run(xT, tabL, pos_table)
    out5 = out4.reshape(MAXLEN, EB, NW, 8, 128)
    return out5.transpose(2, 4, 0, 1, 3).reshape(BATCH, MAXLEN, EMBED)
